# baseline (speedup 1.0000x reference)
import functools

import jax
import jax.numpy as jnp
from jax import lax
from jax.experimental import pallas as pl
from jax.experimental.pallas import tpu as pltpu
from jax.experimental.pallas import tpu_sc as plsc

N_NODES = 10000
D_IN = 128
N_EDGES = 320000
E_TOT = N_EDGES + N_NODES
NC, NS, LANES = 2, 16, 16
NW = NC * NS
CHUNK = 128
CHUNKS_PER_TILE = 2 * (-(-E_TOT // (NW * CHUNK * 2)))
E_PAD = NW * CHUNK * CHUNKS_PER_TILE
N_ACC = 10240
ROWS_PER_TILE = N_ACC // NS
BLK = 400


def _bcast_reg(v, k, heads8):
    lane = lax.iota(jnp.int32, 16)
    idx = (lane // 8) + (2 * k) if heads8 else lane * 0
    dn = lax.GatherDimensionNumbers(
        offset_dims=(), collapsed_slice_dims=(0,), start_index_map=(0,))
    return lax.gather(v, idx[:, None], dn, (1,),
                      mode=lax.GatherScatterMode.PROMISE_IN_BOUNDS)


@functools.lru_cache(maxsize=None)
def _make_edge_kernel(heads8):
    mesh = plsc.VectorSubcoreMesh(
        core_axis_name="c", subcore_axis_name="s", num_cores=NC, num_subcores=NS
    )

    @functools.partial(
        pl.kernel,
        out_type=jax.ShapeDtypeStruct((NC, N_ACC, 80), jnp.float32),
        mesh=mesh,
        compiler_params=pltpu.CompilerParams(
            use_tc_tiling_on_sc=False, needs_layout_passes=False),
        scratch_types=[
            pltpu.VMEM((CHUNK,), jnp.int32),
            pltpu.VMEM((CHUNK,), jnp.int32),
            pltpu.VMEM((CHUNK, 80), jnp.float32),
            pltpu.VMEM((CHUNK, 16), jnp.float32),
            pltpu.VMEM((CHUNK, 80), jnp.float32),
            pltpu.VMEM_SHARED((N_ACC, 80), jnp.float32),
            pltpu.SemaphoreType.DMA,
            pltpu.SemaphoreType.DMA,
        ],
    )
    def edge_kernel(g_hbm, d_hbm, src_hbm, dst_hbm, acc_hbm,
                    idx_s, idx_d, rows_s, rows_d, msg, acc_sh, sem_g, sem_d):
        cid = lax.axis_index("c")
        sid = lax.axis_index("s")
        wid = sid * NC + cid

        def zrow(r, _):
            for k in range(5):
                msg[r, pl.ds(16 * k, 16)] = jnp.zeros((16,), jnp.float32)
            return 0
        lax.fori_loop(0, CHUNK, zrow, 0)
        row0 = sid * ROWS_PER_TILE
        for j in range(ROWS_PER_TILE // CHUNK):
            pltpu.sync_copy(msg, acc_sh.at[pl.ds(row0 + j * CHUNK, CHUNK)])
        plsc.subcore_barrier()

        ebase = wid * (CHUNKS_PER_TILE * CHUNK)

        def chunk(i, _):
            off = ebase + i * CHUNK
            pltpu.sync_copy(src_hbm.at[pl.ds(off, CHUNK)], idx_s)
            pltpu.sync_copy(dst_hbm.at[pl.ds(off, CHUNK)], idx_d)
            cp_s = pltpu.async_copy(g_hbm.at[idx_s], rows_s, sem_g)
            cp_d = pltpu.async_copy(d_hbm.at[idx_d], rows_d, sem_d)
            cp_s.wait()
            cp_d.wait()

            @plsc.parallel_loop(0, CHUNK, step=1, unroll=4)
            def _edges(e):
                va = rows_s[e, pl.ds(64, 16)]
                vd = rows_d[e, pl.ds(0, 16)]
                z = va + vd
                z = jnp.where(z > 0, z, z * jnp.float32(0.2))
                sv = jnp.exp(z)
                msg[e, pl.ds(64, 16)] = sv
                for k in range(4):
                    bk = _bcast_reg(sv, k, heads8)
                    msg[e, pl.ds(16 * k, 16)] = rows_s[e, pl.ds(16 * k, 16)] * bk
            pltpu.sync_copy(msg, acc_sh.at[idx_d], add=True)
            return 0
        lax.fori_loop(0, CHUNKS_PER_TILE, chunk, 0)
        plsc.subcore_barrier()

        for j in range(ROWS_PER_TILE // CHUNK):
            r = row0 + j * CHUNK
            pltpu.sync_copy(acc_sh.at[pl.ds(r, CHUNK)], msg)
            pltpu.sync_copy(msg, acc_hbm.at[cid, pl.ds(r, CHUNK)])

    return edge_kernel


def _mm_body(x_ref, w_ref, o_ref):
    o_ref[...] = jnp.dot(x_ref[...], w_ref[...],
                         preferred_element_type=jnp.float32)


def _ep1_body(acc_ref, e8_ref, b1_ref, w_ref, o_ref):
    o = acc_ref[0] + acc_ref[1]
    m = o[:, :64]
    den = o[:, 64:72]
    den64 = jnp.dot(den, e8_ref[...], preferred_element_type=jnp.float32)
    hr = jnp.maximum(m / (den64 + jnp.float32(1e-16)) + b1_ref[...], 0.0)
    o_ref[...] = jnp.dot(hr, w_ref[...], preferred_element_type=jnp.float32)


def _fin_body(acc_ref, b2_ref, o_ref):
    o = acc_ref[0] + acc_ref[1]
    logits = o[:, :64] / (o[:, 64:65] + jnp.float32(1e-16)) + b2_ref[...]
    mx = jnp.max(logits, axis=1, keepdims=True)
    z = logits - mx
    o_ref[...] = z - jnp.log(jnp.sum(jnp.exp(z), axis=1, keepdims=True))


def kernel(x, edge_index, W1, a_src1, a_dst1, b1, W2, a_src2, a_dst2, b2):
    f32 = jnp.float32
    As = (jnp.eye(8, dtype=f32)[:, None, :] * a_src1[0][:, :, None]).reshape(64, 8)
    Ad = (jnp.eye(8, dtype=f32)[:, None, :] * a_dst1[0][:, :, None]).reshape(64, 8)
    W1s = W1 @ As
    W1d = W1 @ Ad
    Wall1 = jnp.concatenate(
        [W1, W1s, W1d, W1d, jnp.zeros((D_IN, 8), f32)], axis=1)
    W2s = (W2 @ a_src2[0, 0])[:, None]
    W2d = (W2 @ a_dst2[0, 0])[:, None]
    Wall2 = jnp.concatenate(
        [W2, W2s, jnp.zeros((64, 15), f32), W2d, jnp.zeros((64, 15), f32)],
        axis=1)
    E8 = jnp.repeat(jnp.eye(8, dtype=f32), 8, axis=1)

    loop = jnp.arange(N_NODES, dtype=jnp.int32)
    pad = E_PAD - E_TOT
    src = jnp.concatenate([edge_index[0], loop,
                           jnp.zeros((pad,), jnp.int32)])
    dst = jnp.concatenate([edge_index[1], loop,
                           jnp.full((pad,), N_NODES, jnp.int32)])

    nblk = N_NODES // BLK

    P1 = pl.pallas_call(
        _mm_body,
        grid=(nblk,),
        in_specs=[pl.BlockSpec((BLK, D_IN), lambda i: (i, 0)),
                  pl.BlockSpec((D_IN, 96), lambda i: (0, 0))],
        out_specs=pl.BlockSpec((BLK, 96), lambda i: (i, 0)),
        out_shape=jax.ShapeDtypeStruct((N_NODES, 96), f32),
    )(x, Wall1)
    G1 = P1[:, :80]
    D1 = jnp.pad(P1[:, 80:96], ((0, 16), (0, 0)))

    acc1 = _make_edge_kernel(True)(G1, D1, src, dst)

    P2 = pl.pallas_call(
        _ep1_body,
        grid=(nblk,),
        in_specs=[pl.BlockSpec((NC, BLK, 80), lambda i: (0, i, 0)),
                  pl.BlockSpec((8, 64), lambda i: (0, 0)),
                  pl.BlockSpec((1, 64), lambda i: (0, 0)),
                  pl.BlockSpec((64, 96), lambda i: (0, 0))],
        out_specs=pl.BlockSpec((BLK, 96), lambda i: (i, 0)),
        out_shape=jax.ShapeDtypeStruct((N_NODES, 96), f32),
    )(acc1, E8, b1.reshape(1, 64), Wall2)
    G2 = P2[:, :80]
    D2 = jnp.pad(P2[:, 80:96], ((0, 16), (0, 0)))

    acc2 = _make_edge_kernel(False)(G2, D2, src, dst)

    out = pl.pallas_call(
        _fin_body,
        grid=(nblk,),
        in_specs=[pl.BlockSpec((NC, BLK, 80), lambda i: (0, i, 0)),
                  pl.BlockSpec((1, 64), lambda i: (0, 0))],
        out_specs=pl.BlockSpec((BLK, 64), lambda i: (i, 0)),
        out_shape=jax.ShapeDtypeStruct((N_NODES, 64), f32),
    )(acc2, b2.reshape(1, 64))
    return out


# speedup vs baseline: 1.0080x; 1.0080x over previous
import functools

import jax
import jax.numpy as jnp
from jax import lax
from jax.experimental import pallas as pl
from jax.experimental.pallas import tpu as pltpu
from jax.experimental.pallas import tpu_sc as plsc

N_NODES = 10000
D_IN = 128
N_EDGES = 320000
E_TOT = N_EDGES + N_NODES
NC, NS, LANES = 2, 16, 16
NW = NC * NS
CHUNK = 128
CHUNKS_PER_TILE = 2 * (-(-E_TOT // (NW * CHUNK * 2)))
E_PAD = NW * CHUNK * CHUNKS_PER_TILE
N_ACC = 10240
ROWS_PER_TILE = N_ACC // NS
BLK = 400


def _bcast_reg(v, k, heads8):
    lane = lax.iota(jnp.int32, 16)
    idx = (lane // 8) + (2 * k) if heads8 else lane * 0
    dn = lax.GatherDimensionNumbers(
        offset_dims=(), collapsed_slice_dims=(0,), start_index_map=(0,))
    return lax.gather(v, idx[:, None], dn, (1,),
                      mode=lax.GatherScatterMode.PROMISE_IN_BOUNDS)


@functools.lru_cache(maxsize=None)
def _make_edge_kernel(heads8):
    mesh = plsc.VectorSubcoreMesh(
        core_axis_name="c", subcore_axis_name="s", num_cores=NC, num_subcores=NS
    )

    @functools.partial(
        pl.kernel,
        out_type=jax.ShapeDtypeStruct((NC, N_ACC, 80), jnp.float32),
        mesh=mesh,
        compiler_params=pltpu.CompilerParams(
            use_tc_tiling_on_sc=False, needs_layout_passes=False),
        scratch_types=[
            pltpu.VMEM((CHUNK,), jnp.int32),
            pltpu.VMEM((CHUNK,), jnp.int32),
            pltpu.VMEM((CHUNK, 80), jnp.float32),
            pltpu.VMEM((CHUNK, 16), jnp.float32),
            pltpu.VMEM((CHUNK, 80), jnp.float32),
            pltpu.VMEM_SHARED((N_ACC, 80), jnp.float32),
            pltpu.SemaphoreType.DMA,
            pltpu.SemaphoreType.DMA,
        ],
    )
    def edge_kernel(g_hbm, d_hbm, src_hbm, dst_hbm, acc_hbm,
                    idx_s, idx_d, rows_s, rows_d, msg, acc_sh, sem_g, sem_d):
        cid = lax.axis_index("c")
        sid = lax.axis_index("s")
        wid = sid * NC + cid

        def zrow(r, _):
            for k in range(5):
                msg[r, pl.ds(16 * k, 16)] = jnp.zeros((16,), jnp.float32)
            return 0
        lax.fori_loop(0, CHUNK, zrow, 0)
        row0 = sid * ROWS_PER_TILE
        for j in range(ROWS_PER_TILE // CHUNK):
            pltpu.sync_copy(msg, acc_sh.at[pl.ds(row0 + j * CHUNK, CHUNK)])
        plsc.subcore_barrier()

        ebase = wid * (CHUNKS_PER_TILE * CHUNK)

        def chunk(i, _):
            off = ebase + i * CHUNK
            pltpu.sync_copy(src_hbm.at[pl.ds(off, CHUNK)], idx_s)
            pltpu.sync_copy(dst_hbm.at[pl.ds(off, CHUNK)], idx_d)
            cp_s = pltpu.async_copy(g_hbm.at[idx_s], rows_s, sem_g)
            cp_d = pltpu.async_copy(d_hbm.at[idx_d], rows_d, sem_d)
            cp_s.wait()
            cp_d.wait()

            @plsc.parallel_loop(0, CHUNK, step=1, unroll=4)
            def _edges(e):
                va = rows_s[e, pl.ds(64, 16)]
                vd = rows_d[e, pl.ds(0, 16)]
                z = va + vd
                z = jnp.where(z > 0, z, z * jnp.float32(0.2))
                sv = jnp.exp(z)
                msg[e, pl.ds(64, 16)] = sv
                for k in range(4):
                    bk = _bcast_reg(sv, k, heads8)
                    msg[e, pl.ds(16 * k, 16)] = rows_s[e, pl.ds(16 * k, 16)] * bk
            pltpu.sync_copy(msg, acc_sh.at[idx_d], add=True)
            return 0
        lax.fori_loop(0, CHUNKS_PER_TILE, chunk, 0)
        plsc.subcore_barrier()

        for j in range(ROWS_PER_TILE // CHUNK):
            r = row0 + j * CHUNK
            pltpu.sync_copy(acc_sh.at[pl.ds(r, CHUNK)], msg)
            pltpu.sync_copy(msg, acc_hbm.at[cid, pl.ds(r, CHUNK)])

    return edge_kernel


def _mm_body(x_ref, w_ref, o_ref):
    o_ref[...] = jnp.dot(x_ref[...], w_ref[...],
                         preferred_element_type=jnp.float32)


def _ep1_body(acc_ref, e8_ref, b1_ref, w_ref, o_ref):
    o = acc_ref[0] + acc_ref[1]
    m = o[:, :64]
    den = o[:, 64:72]
    den64 = jnp.dot(den, e8_ref[...], preferred_element_type=jnp.float32)
    hr = jnp.maximum(m / (den64 + jnp.float32(1e-16)) + b1_ref[...], 0.0)
    o_ref[...] = jnp.dot(hr, w_ref[...], preferred_element_type=jnp.float32)


def _fin_body(acc_ref, b2_ref, o_ref):
    o = acc_ref[0] + acc_ref[1]
    logits = o[:, :64] / (o[:, 64:65] + jnp.float32(1e-16)) + b2_ref[...]
    mx = jnp.max(logits, axis=1, keepdims=True)
    z = logits - mx
    o_ref[...] = z - jnp.log(jnp.sum(jnp.exp(z), axis=1, keepdims=True))


def kernel(x, edge_index, W1, a_src1, a_dst1, b1, W2, a_src2, a_dst2, b2):
    f32 = jnp.float32
    As = (jnp.eye(8, dtype=f32)[:, None, :] * a_src1[0][:, :, None]).reshape(64, 8)
    Ad = (jnp.eye(8, dtype=f32)[:, None, :] * a_dst1[0][:, :, None]).reshape(64, 8)
    W1s = W1 @ As
    W1d = W1 @ Ad
    Wall1 = jnp.concatenate(
        [W1, W1s, W1d, W1d, jnp.zeros((D_IN, 8), f32)], axis=1)
    W2s = (W2 @ a_src2[0, 0])[:, None]
    W2d = (W2 @ a_dst2[0, 0])[:, None]
    Wall2 = jnp.concatenate(
        [W2, W2s, jnp.zeros((64, 15), f32), W2d, jnp.zeros((64, 15), f32)],
        axis=1)
    E8 = jnp.repeat(jnp.eye(8, dtype=f32), 8, axis=1)

    loop = jnp.arange(N_NODES, dtype=jnp.int32)
    pad = E_PAD - E_TOT
    pad_dst = N_NODES + (jnp.arange(pad, dtype=jnp.int32) % (N_ACC - N_NODES))
    src = jnp.concatenate([edge_index[0], loop,
                           jnp.zeros((pad,), jnp.int32)])
    dst = jnp.concatenate([edge_index[1], loop, pad_dst])

    nblk = N_NODES // BLK

    P1 = pl.pallas_call(
        _mm_body,
        grid=(nblk,),
        in_specs=[pl.BlockSpec((BLK, D_IN), lambda i: (i, 0)),
                  pl.BlockSpec((D_IN, 96), lambda i: (0, 0))],
        out_specs=pl.BlockSpec((BLK, 96), lambda i: (i, 0)),
        out_shape=jax.ShapeDtypeStruct((N_NODES, 96), f32),
    )(x, Wall1)
    G1 = P1[:, :80]
    D1 = jnp.pad(P1[:, 80:96], ((0, N_ACC - N_NODES), (0, 0)))

    acc1 = _make_edge_kernel(True)(G1, D1, src, dst)

    P2 = pl.pallas_call(
        _ep1_body,
        grid=(nblk,),
        in_specs=[pl.BlockSpec((NC, BLK, 80), lambda i: (0, i, 0)),
                  pl.BlockSpec((8, 64), lambda i: (0, 0)),
                  pl.BlockSpec((1, 64), lambda i: (0, 0)),
                  pl.BlockSpec((64, 96), lambda i: (0, 0))],
        out_specs=pl.BlockSpec((BLK, 96), lambda i: (i, 0)),
        out_shape=jax.ShapeDtypeStruct((N_NODES, 96), f32),
    )(acc1, E8, b1.reshape(1, 64), Wall2)
    G2 = P2[:, :80]
    D2 = jnp.pad(P2[:, 80:96], ((0, N_ACC - N_NODES), (0, 0)))

    acc2 = _make_edge_kernel(False)(G2, D2, src, dst)

    out = pl.pallas_call(
        _fin_body,
        grid=(nblk,),
        in_specs=[pl.BlockSpec((NC, BLK, 80), lambda i: (0, i, 0)),
                  pl.BlockSpec((1, 64), lambda i: (0, 0))],
        out_specs=pl.BlockSpec((BLK, 64), lambda i: (i, 0)),
        out_shape=jax.ShapeDtypeStruct((N_NODES, 64), f32),
    )(acc2, b2.reshape(1, 64))
    return out


# speedup vs baseline: 1.3327x; 1.3221x over previous
import functools

import jax
import jax.numpy as jnp
from jax import lax
from jax.experimental import pallas as pl
from jax.experimental.pallas import tpu as pltpu
from jax.experimental.pallas import tpu_sc as plsc

N_NODES = 10000
D_IN = 128
N_EDGES = 320000
E_TOT = N_EDGES + N_NODES
NC, NS, LANES = 2, 16, 16
NW = NC * NS
CHUNK = 128
CHUNKS_PER_TILE = -(-E_TOT // (NW * CHUNK))
E_PAD = NW * CHUNK * CHUNKS_PER_TILE
N_ACC = 10240
ROWS_PER_TILE = N_ACC // NS
BLK = 400


def _bcast_reg(v, k, heads8):
    lane = lax.iota(jnp.int32, 16)
    idx = (lane // 8) + (2 * k) if heads8 else lane * 0
    dn = lax.GatherDimensionNumbers(
        offset_dims=(), collapsed_slice_dims=(0,), start_index_map=(0,))
    return lax.gather(v, idx[:, None], dn, (1,),
                      mode=lax.GatherScatterMode.PROMISE_IN_BOUNDS)


@functools.lru_cache(maxsize=None)
def _make_edge_kernel(heads8):
    mesh = plsc.VectorSubcoreMesh(
        core_axis_name="c", subcore_axis_name="s", num_cores=NC, num_subcores=NS
    )

    @functools.partial(
        pl.kernel,
        out_type=jax.ShapeDtypeStruct((NC, N_ACC, 80), jnp.float32),
        mesh=mesh,
        compiler_params=pltpu.CompilerParams(
            use_tc_tiling_on_sc=False, needs_layout_passes=False),
        scratch_types=[
            pltpu.VMEM((CHUNK,), jnp.int32),
            pltpu.VMEM((CHUNK,), jnp.int32),
            pltpu.VMEM((CHUNK, 80), jnp.float32),
            pltpu.VMEM((CHUNK, 16), jnp.float32),
            pltpu.VMEM((CHUNK, 80), jnp.float32),
            pltpu.VMEM_SHARED((N_ACC, 80), jnp.float32),
            pltpu.SemaphoreType.DMA,
            pltpu.SemaphoreType.DMA,
        ],
    )
    def edge_kernel(g_hbm, d_hbm, src_hbm, dst_hbm, acc_hbm,
                    idx_s, idx_d, rows_s, rows_d, msg, acc_sh, sem_g, sem_d):
        cid = lax.axis_index("c")
        sid = lax.axis_index("s")
        wid = sid * NC + cid

        def zrow(r, _):
            for k in range(5):
                msg[r, pl.ds(16 * k, 16)] = jnp.zeros((16,), jnp.float32)
            return 0
        lax.fori_loop(0, CHUNK, zrow, 0)
        row0 = sid * ROWS_PER_TILE
        for j in range(ROWS_PER_TILE // CHUNK):
            pltpu.sync_copy(msg, acc_sh.at[pl.ds(row0 + j * CHUNK, CHUNK)])
        plsc.subcore_barrier()

        ebase = wid * (CHUNKS_PER_TILE * CHUNK)

        def chunk(i, _):
            off = ebase + i * CHUNK
            pltpu.sync_copy(src_hbm.at[pl.ds(off, CHUNK)], idx_s)
            pltpu.sync_copy(dst_hbm.at[pl.ds(off, CHUNK)], idx_d)
            cp_s = pltpu.async_copy(g_hbm.at[idx_s], rows_s, sem_g)
            cp_d = pltpu.async_copy(d_hbm.at[idx_d], rows_d, sem_d)
            cp_s.wait()
            cp_d.wait()

            @plsc.parallel_loop(0, CHUNK, step=1, unroll=4)
            def _edges(e):
                va = rows_s[e, pl.ds(64, 16)]
                vd = rows_d[e, pl.ds(0, 16)]
                z = va + vd
                z = jnp.where(z > 0, z, z * jnp.float32(0.2))
                sv = jnp.exp(z)
                msg[e, pl.ds(64, 16)] = sv
                for k in range(4):
                    bk = _bcast_reg(sv, k, heads8)
                    msg[e, pl.ds(16 * k, 16)] = rows_s[e, pl.ds(16 * k, 16)] * bk
            pltpu.sync_copy(msg, acc_sh.at[idx_d], add=True)
            return 0
        lax.fori_loop(0, CHUNKS_PER_TILE, chunk, 0)
        plsc.subcore_barrier()

        for j in range(ROWS_PER_TILE // CHUNK):
            r = row0 + j * CHUNK
            pltpu.sync_copy(acc_sh.at[pl.ds(r, CHUNK)], msg)
            pltpu.sync_copy(msg, acc_hbm.at[cid, pl.ds(r, CHUNK)])

    return edge_kernel


def _mm_body(x_ref, w_ref, o_ref):
    o_ref[...] = jnp.dot(x_ref[...], w_ref[...],
                         preferred_element_type=jnp.float32)


def _ep1_body(acc_ref, e8_ref, b1_ref, w_ref, o_ref):
    o = acc_ref[0] + acc_ref[1]
    m = o[:, :64]
    den = o[:, 64:72]
    den64 = jnp.dot(den, e8_ref[...], preferred_element_type=jnp.float32)
    hr = jnp.maximum(m / (den64 + jnp.float32(1e-16)) + b1_ref[...], 0.0)
    o_ref[...] = jnp.dot(hr, w_ref[...], preferred_element_type=jnp.float32)


def _fin_body(acc_ref, b2_ref, o_ref):
    o = acc_ref[0] + acc_ref[1]
    logits = o[:, :64] / (o[:, 64:65] + jnp.float32(1e-16)) + b2_ref[...]
    mx = jnp.max(logits, axis=1, keepdims=True)
    z = logits - mx
    o_ref[...] = z - jnp.log(jnp.sum(jnp.exp(z), axis=1, keepdims=True))


def kernel(x, edge_index, W1, a_src1, a_dst1, b1, W2, a_src2, a_dst2, b2):
    f32 = jnp.float32
    As = (jnp.eye(8, dtype=f32)[:, None, :] * a_src1[0][:, :, None]).reshape(64, 8)
    Ad = (jnp.eye(8, dtype=f32)[:, None, :] * a_dst1[0][:, :, None]).reshape(64, 8)
    W1s = W1 @ As
    W1d = W1 @ Ad
    Wall1 = jnp.concatenate(
        [W1, W1s, W1d, W1d, jnp.zeros((D_IN, 8), f32)], axis=1)
    W2s = (W2 @ a_src2[0, 0])[:, None]
    W2d = (W2 @ a_dst2[0, 0])[:, None]
    Wall2 = jnp.concatenate(
        [W2, W2s, jnp.zeros((64, 15), f32), W2d, jnp.zeros((64, 15), f32)],
        axis=1)
    E8 = jnp.repeat(jnp.eye(8, dtype=f32), 8, axis=1)

    loop = jnp.arange(N_NODES, dtype=jnp.int32)
    pad = E_PAD - E_TOT
    pad_dst = N_NODES + (jnp.arange(pad, dtype=jnp.int32) % (N_ACC - N_NODES))
    src = jnp.concatenate([edge_index[0], loop,
                           jnp.zeros((pad,), jnp.int32)])
    dst = jnp.concatenate([edge_index[1], loop, pad_dst])

    nblk = N_NODES // BLK

    P1 = pl.pallas_call(
        _mm_body,
        grid=(nblk,),
        in_specs=[pl.BlockSpec((BLK, D_IN), lambda i: (i, 0)),
                  pl.BlockSpec((D_IN, 96), lambda i: (0, 0))],
        out_specs=pl.BlockSpec((BLK, 96), lambda i: (i, 0)),
        out_shape=jax.ShapeDtypeStruct((N_NODES, 96), f32),
    )(x, Wall1)
    G1 = P1[:, :80]
    D1 = jnp.pad(P1[:, 80:96], ((0, N_ACC - N_NODES), (0, 0)))

    acc1 = _make_edge_kernel(True)(G1, D1, src, dst)

    P2 = pl.pallas_call(
        _ep1_body,
        grid=(nblk,),
        in_specs=[pl.BlockSpec((NC, BLK, 80), lambda i: (0, i, 0)),
                  pl.BlockSpec((8, 64), lambda i: (0, 0)),
                  pl.BlockSpec((1, 64), lambda i: (0, 0)),
                  pl.BlockSpec((64, 96), lambda i: (0, 0))],
        out_specs=pl.BlockSpec((BLK, 96), lambda i: (i, 0)),
        out_shape=jax.ShapeDtypeStruct((N_NODES, 96), f32),
    )(acc1, E8, b1.reshape(1, 64), Wall2)
    G2 = P2[:, :80]
    D2 = jnp.pad(P2[:, 80:96], ((0, N_ACC - N_NODES), (0, 0)))

    acc2 = _make_edge_kernel(False)(G2, D2, src, dst)

    out = pl.pallas_call(
        _fin_body,
        grid=(nblk,),
        in_specs=[pl.BlockSpec((NC, BLK, 80), lambda i: (0, i, 0)),
                  pl.BlockSpec((1, 64), lambda i: (0, 0))],
        out_specs=pl.BlockSpec((BLK, 64), lambda i: (i, 0)),
        out_shape=jax.ShapeDtypeStruct((N_NODES, 64), f32),
    )(acc2, b2.reshape(1, 64))
    return out


# speedup vs baseline: 1.4733x; 1.1056x over previous
import functools

import jax
import jax.numpy as jnp
from jax import lax
from jax.experimental import pallas as pl
from jax.experimental.pallas import tpu as pltpu
from jax.experimental.pallas import tpu_sc as plsc

N_NODES = 10000
D_IN = 128
N_EDGES = 320000
E_TOT = N_EDGES + N_NODES
NC, NS, LANES = 2, 16, 16
NW = NC * NS
CHUNK = 128
CHUNKS_PER_TILE = 2 * (-(-E_TOT // (NW * CHUNK * 2)))
E_PAD = NW * CHUNK * CHUNKS_PER_TILE
N_ACC = 10240
ROWS_PER_TILE = N_ACC // NS
BLK = 400


def _bcast_reg(v, k, heads8):
    lane = lax.iota(jnp.int32, 16)
    idx = (lane // 8) + (2 * k) if heads8 else lane * 0
    dn = lax.GatherDimensionNumbers(
        offset_dims=(), collapsed_slice_dims=(0,), start_index_map=(0,))
    return lax.gather(v, idx[:, None], dn, (1,),
                      mode=lax.GatherScatterMode.PROMISE_IN_BOUNDS)


@functools.lru_cache(maxsize=None)
def _make_edge_kernel(heads8):
    mesh = plsc.VectorSubcoreMesh(
        core_axis_name="c", subcore_axis_name="s", num_cores=NC, num_subcores=NS
    )

    @functools.partial(
        pl.kernel,
        out_type=jax.ShapeDtypeStruct((NC, N_ACC, 80), jnp.float32),
        mesh=mesh,
        compiler_params=pltpu.CompilerParams(
            use_tc_tiling_on_sc=False, needs_layout_passes=False),
        scratch_types=[
            pltpu.VMEM((CHUNK,), jnp.int32),
            pltpu.VMEM((CHUNK,), jnp.int32),
            pltpu.VMEM((CHUNK, 80), jnp.float32),
            pltpu.VMEM((CHUNK, 16), jnp.float32),
            pltpu.VMEM((CHUNK, 80), jnp.float32),
            pltpu.VMEM_SHARED((N_ACC, 80), jnp.float32),
            pltpu.SemaphoreType.DMA,
            pltpu.SemaphoreType.DMA,
        ],
    )
    def edge_kernel(g_hbm, d_hbm, src_hbm, dst_hbm, acc_hbm,
                    idx_s, idx_d, rows_s, rows_d, msg, acc_sh, sem_g, sem_d):
        cid = lax.axis_index("c")
        sid = lax.axis_index("s")
        wid = sid * NC + cid

        def zrow(r, _):
            for k in range(5):
                msg[r, pl.ds(16 * k, 16)] = jnp.zeros((16,), jnp.float32)
            return 0
        lax.fori_loop(0, CHUNK, zrow, 0)
        row0 = sid * ROWS_PER_TILE
        for j in range(ROWS_PER_TILE // CHUNK):
            pltpu.sync_copy(msg, acc_sh.at[pl.ds(row0 + j * CHUNK, CHUNK)])
        plsc.subcore_barrier()

        ebase = wid * (CHUNKS_PER_TILE * CHUNK)

        def chunk(i, _):
            off = ebase + i * CHUNK
            pltpu.sync_copy(src_hbm.at[pl.ds(off, CHUNK)], idx_s)
            pltpu.sync_copy(dst_hbm.at[pl.ds(off, CHUNK)], idx_d)
            cp_s = pltpu.async_copy(g_hbm.at[idx_s], rows_s, sem_g)
            cp_d = pltpu.async_copy(d_hbm.at[idx_d], rows_d, sem_d)
            cp_s.wait()
            cp_d.wait()

            @plsc.parallel_loop(0, CHUNK, step=1, unroll=4)
            def _edges(e):
                va = rows_s[e, pl.ds(64, 16)]
                vd = rows_d[e, pl.ds(0, 16)]
                z = va + vd
                z = jnp.where(z > 0, z, z * jnp.float32(0.2))
                sv = jnp.exp(z)
                msg[e, pl.ds(64, 16)] = sv
                for k in range(4):
                    bk = _bcast_reg(sv, k, heads8)
                    msg[e, pl.ds(16 * k, 16)] = rows_s[e, pl.ds(16 * k, 16)] * bk
            pltpu.sync_copy(msg, acc_sh.at[idx_d], add=True)
            return 0
        lax.fori_loop(0, CHUNKS_PER_TILE, chunk, 0)
        plsc.subcore_barrier()

        for j in range(ROWS_PER_TILE // CHUNK):
            r = row0 + j * CHUNK
            pltpu.sync_copy(acc_sh.at[pl.ds(r, CHUNK)], msg)
            pltpu.sync_copy(msg, acc_hbm.at[cid, pl.ds(r, CHUNK)])

    return edge_kernel


def _mm_body(x_ref, w_ref, o_ref):
    o_ref[...] = jnp.dot(x_ref[...], w_ref[...],
                         preferred_element_type=jnp.float32)


def _ep1_body(acc_ref, e8_ref, b1_ref, w_ref, o_ref):
    o = acc_ref[0] + acc_ref[1]
    m = o[:, :64]
    den = o[:, 64:72]
    den64 = jnp.dot(den, e8_ref[...], preferred_element_type=jnp.float32)
    hr = jnp.maximum(m / (den64 + jnp.float32(1e-16)) + b1_ref[...], 0.0)
    o_ref[...] = jnp.dot(hr, w_ref[...], preferred_element_type=jnp.float32)


def _fin_body(acc_ref, b2_ref, o_ref):
    o = acc_ref[0] + acc_ref[1]
    logits = o[:, :64] / (o[:, 64:65] + jnp.float32(1e-16)) + b2_ref[...]
    mx = jnp.max(logits, axis=1, keepdims=True)
    z = logits - mx
    o_ref[...] = z - jnp.log(jnp.sum(jnp.exp(z), axis=1, keepdims=True))


def kernel(x, edge_index, W1, a_src1, a_dst1, b1, W2, a_src2, a_dst2, b2):
    f32 = jnp.float32
    As = (jnp.eye(8, dtype=f32)[:, None, :] * a_src1[0][:, :, None]).reshape(64, 8)
    Ad = (jnp.eye(8, dtype=f32)[:, None, :] * a_dst1[0][:, :, None]).reshape(64, 8)
    W1s = W1 @ As
    W1d = W1 @ Ad
    Wall1 = jnp.concatenate(
        [W1, W1s, W1d, W1d, jnp.zeros((D_IN, 8), f32)], axis=1)
    W2s = (W2 @ a_src2[0, 0])[:, None]
    W2d = (W2 @ a_dst2[0, 0])[:, None]
    Wall2 = jnp.concatenate(
        [W2, W2s, jnp.zeros((64, 15), f32), W2d, jnp.zeros((64, 15), f32)],
        axis=1)
    E8 = jnp.repeat(jnp.eye(8, dtype=f32), 8, axis=1)

    loop = jnp.arange(N_NODES, dtype=jnp.int32)
    pad = E_PAD - E_TOT
    pad_dst = N_NODES + (jnp.arange(pad, dtype=jnp.int32) % (N_ACC - N_NODES))
    pad_src = jnp.arange(pad, dtype=jnp.int32) % N_NODES
    src = jnp.concatenate([edge_index[0], loop, pad_src])
    dst = jnp.concatenate([edge_index[1], loop, pad_dst])

    nblk = N_NODES // BLK

    P1 = pl.pallas_call(
        _mm_body,
        grid=(nblk,),
        in_specs=[pl.BlockSpec((BLK, D_IN), lambda i: (i, 0)),
                  pl.BlockSpec((D_IN, 96), lambda i: (0, 0))],
        out_specs=pl.BlockSpec((BLK, 96), lambda i: (i, 0)),
        out_shape=jax.ShapeDtypeStruct((N_NODES, 96), f32),
    )(x, Wall1)
    G1 = P1[:, :80]
    D1 = jnp.pad(P1[:, 80:96], ((0, N_ACC - N_NODES), (0, 0)))

    acc1 = _make_edge_kernel(True)(G1, D1, src, dst)

    P2 = pl.pallas_call(
        _ep1_body,
        grid=(nblk,),
        in_specs=[pl.BlockSpec((NC, BLK, 80), lambda i: (0, i, 0)),
                  pl.BlockSpec((8, 64), lambda i: (0, 0)),
                  pl.BlockSpec((1, 64), lambda i: (0, 0)),
                  pl.BlockSpec((64, 96), lambda i: (0, 0))],
        out_specs=pl.BlockSpec((BLK, 96), lambda i: (i, 0)),
        out_shape=jax.ShapeDtypeStruct((N_NODES, 96), f32),
    )(acc1, E8, b1.reshape(1, 64), Wall2)
    G2 = P2[:, :80]
    D2 = jnp.pad(P2[:, 80:96], ((0, N_ACC - N_NODES), (0, 0)))

    acc2 = _make_edge_kernel(False)(G2, D2, src, dst)

    out = pl.pallas_call(
        _fin_body,
        grid=(nblk,),
        in_specs=[pl.BlockSpec((NC, BLK, 80), lambda i: (0, i, 0)),
                  pl.BlockSpec((1, 64), lambda i: (0, 0))],
        out_specs=pl.BlockSpec((BLK, 64), lambda i: (i, 0)),
        out_shape=jax.ShapeDtypeStruct((N_NODES, 64), f32),
    )(acc2, b2.reshape(1, 64))
    return out


# speedup vs baseline: 1.8701x; 1.2693x over previous
import functools

import jax
import jax.numpy as jnp
from jax import lax
from jax.experimental import pallas as pl
from jax.experimental.pallas import tpu as pltpu
from jax.experimental.pallas import tpu_sc as plsc

N_NODES = 10000
D_IN = 128
N_EDGES = 320000
E_TOT = N_EDGES + N_NODES
NC, NS, LANES = 2, 16, 16
NW = NC * NS
CHUNK = 128
CHUNKS_PER_TILE = 2 * (-(-E_TOT // (NW * CHUNK * 2)))
E_PAD = NW * CHUNK * CHUNKS_PER_TILE
N_ACC = 10240
ROWS_PER_TILE = N_ACC // NS
BLK = 400


def _bcast_reg(v, k, heads8):
    lane = lax.iota(jnp.int32, 16)
    idx = (lane // 8) + (2 * k) if heads8 else lane * 0
    dn = lax.GatherDimensionNumbers(
        offset_dims=(), collapsed_slice_dims=(0,), start_index_map=(0,))
    return lax.gather(v, idx[:, None], dn, (1,),
                      mode=lax.GatherScatterMode.PROMISE_IN_BOUNDS)


@functools.lru_cache(maxsize=None)
def _make_edge_kernel(heads8):
    mesh = plsc.VectorSubcoreMesh(
        core_axis_name="c", subcore_axis_name="s", num_cores=NC, num_subcores=NS
    )

    @functools.partial(
        pl.kernel,
        out_type=jax.ShapeDtypeStruct((NC, N_ACC, 80), jnp.float32),
        mesh=mesh,
        compiler_params=pltpu.CompilerParams(
            use_tc_tiling_on_sc=False, needs_layout_passes=False),
        scratch_types=[
            pltpu.VMEM((2, CHUNK), jnp.int32),
            pltpu.VMEM((2, CHUNK), jnp.int32),
            pltpu.VMEM((2, CHUNK, 80), jnp.float32),
            pltpu.VMEM((2, CHUNK, 16), jnp.float32),
            pltpu.VMEM((CHUNK, 80), jnp.float32),
            pltpu.VMEM_SHARED((N_ACC, 80), jnp.float32),
            pltpu.SemaphoreType.DMA,
            pltpu.SemaphoreType.DMA,
            pltpu.SemaphoreType.DMA,
            pltpu.SemaphoreType.DMA,
        ],
    )
    def edge_kernel(g_hbm, d_hbm, src_hbm, dst_hbm, acc_hbm,
                    idx_s, idx_d, rows_s, rows_d, msg, acc_sh,
                    sem_s0, sem_s1, sem_d0, sem_d1):
        cid = lax.axis_index("c")
        sid = lax.axis_index("s")
        wid = sid * NC + cid
        sem_s = (sem_s0, sem_s1)
        sem_d = (sem_d0, sem_d1)
        ebase = wid * (CHUNKS_PER_TILE * CHUNK)

        def issue(g, b):
            off = ebase + g * CHUNK
            pltpu.sync_copy(src_hbm.at[pl.ds(off, CHUNK)], idx_s.at[b])
            pltpu.sync_copy(dst_hbm.at[pl.ds(off, CHUNK)], idx_d.at[b])
            pltpu.async_copy(g_hbm.at[idx_s.at[b]], rows_s.at[b], sem_s[b])
            pltpu.async_copy(d_hbm.at[idx_d.at[b]], rows_d.at[b], sem_d[b])

        issue(0, 0)

        def zrow(r, _):
            for k in range(5):
                msg[r, pl.ds(16 * k, 16)] = jnp.zeros((16,), jnp.float32)
            return 0
        lax.fori_loop(0, CHUNK, zrow, 0)
        row0 = sid * ROWS_PER_TILE
        for j in range(ROWS_PER_TILE // CHUNK):
            pltpu.sync_copy(msg, acc_sh.at[pl.ds(row0 + j * CHUNK, CHUNK)])
        plsc.subcore_barrier()

        def pair(i, _):
            for b in range(2):
                g = 2 * i + b
                pltpu.make_async_copy(
                    g_hbm.at[idx_s.at[b]], rows_s.at[b], sem_s[b]).wait()
                pltpu.make_async_copy(
                    d_hbm.at[idx_d.at[b]], rows_d.at[b], sem_d[b]).wait()

                @pl.when(g + 1 < CHUNKS_PER_TILE)
                def _():
                    issue(g + 1, 1 - b)

                rs = rows_s.at[b]

                @plsc.parallel_loop(0, CHUNK, step=1, unroll=4)
                def _edges(e):
                    va = rs[e, pl.ds(64, 16)]
                    vd = rows_d[b, e, pl.ds(0, 16)]
                    z = va + vd
                    z = jnp.where(z > 0, z, z * jnp.float32(0.2))
                    sv = jnp.exp(z)
                    msg[e, pl.ds(64, 16)] = sv
                    for k in range(4):
                        bk = _bcast_reg(sv, k, heads8)
                        msg[e, pl.ds(16 * k, 16)] = rs[e, pl.ds(16 * k, 16)] * bk
                pltpu.sync_copy(msg, acc_sh.at[idx_d.at[b]], add=True)
            return 0
        lax.fori_loop(0, CHUNKS_PER_TILE // 2, pair, 0)
        plsc.subcore_barrier()

        for j in range(ROWS_PER_TILE // CHUNK):
            r = row0 + j * CHUNK
            pltpu.sync_copy(acc_sh.at[pl.ds(r, CHUNK)], msg)
            pltpu.sync_copy(msg, acc_hbm.at[cid, pl.ds(r, CHUNK)])

    return edge_kernel


def _mm_body(x_ref, w_ref, o_ref):
    o_ref[...] = jnp.dot(x_ref[...], w_ref[...],
                         preferred_element_type=jnp.float32)


def _ep1_body(acc_ref, e8_ref, b1_ref, w_ref, o_ref):
    o = acc_ref[0] + acc_ref[1]
    m = o[:, :64]
    den = o[:, 64:72]
    den64 = jnp.dot(den, e8_ref[...], preferred_element_type=jnp.float32)
    hr = jnp.maximum(m / (den64 + jnp.float32(1e-16)) + b1_ref[...], 0.0)
    o_ref[...] = jnp.dot(hr, w_ref[...], preferred_element_type=jnp.float32)


def _fin_body(acc_ref, b2_ref, o_ref):
    o = acc_ref[0] + acc_ref[1]
    logits = o[:, :64] / (o[:, 64:65] + jnp.float32(1e-16)) + b2_ref[...]
    mx = jnp.max(logits, axis=1, keepdims=True)
    z = logits - mx
    o_ref[...] = z - jnp.log(jnp.sum(jnp.exp(z), axis=1, keepdims=True))


def kernel(x, edge_index, W1, a_src1, a_dst1, b1, W2, a_src2, a_dst2, b2):
    f32 = jnp.float32
    As = (jnp.eye(8, dtype=f32)[:, None, :] * a_src1[0][:, :, None]).reshape(64, 8)
    Ad = (jnp.eye(8, dtype=f32)[:, None, :] * a_dst1[0][:, :, None]).reshape(64, 8)
    W1s = W1 @ As
    W1d = W1 @ Ad
    Wall1 = jnp.concatenate(
        [W1, W1s, W1d, W1d, jnp.zeros((D_IN, 8), f32)], axis=1)
    W2s = (W2 @ a_src2[0, 0])[:, None]
    W2d = (W2 @ a_dst2[0, 0])[:, None]
    Wall2 = jnp.concatenate(
        [W2, W2s, jnp.zeros((64, 15), f32), W2d, jnp.zeros((64, 15), f32)],
        axis=1)
    E8 = jnp.repeat(jnp.eye(8, dtype=f32), 8, axis=1)

    loop = jnp.arange(N_NODES, dtype=jnp.int32)
    pad = E_PAD - E_TOT
    pad_dst = N_NODES + (jnp.arange(pad, dtype=jnp.int32) % (N_ACC - N_NODES))
    pad_src = jnp.arange(pad, dtype=jnp.int32) % N_NODES
    src = jnp.concatenate([edge_index[0], loop, pad_src])
    dst = jnp.concatenate([edge_index[1], loop, pad_dst])

    nblk = N_NODES // BLK

    P1 = pl.pallas_call(
        _mm_body,
        grid=(nblk,),
        in_specs=[pl.BlockSpec((BLK, D_IN), lambda i: (i, 0)),
                  pl.BlockSpec((D_IN, 96), lambda i: (0, 0))],
        out_specs=pl.BlockSpec((BLK, 96), lambda i: (i, 0)),
        out_shape=jax.ShapeDtypeStruct((N_NODES, 96), f32),
    )(x, Wall1)
    G1 = P1[:, :80]
    D1 = jnp.pad(P1[:, 80:96], ((0, N_ACC - N_NODES), (0, 0)))

    acc1 = _make_edge_kernel(True)(G1, D1, src, dst)

    P2 = pl.pallas_call(
        _ep1_body,
        grid=(nblk,),
        in_specs=[pl.BlockSpec((NC, BLK, 80), lambda i: (0, i, 0)),
                  pl.BlockSpec((8, 64), lambda i: (0, 0)),
                  pl.BlockSpec((1, 64), lambda i: (0, 0)),
                  pl.BlockSpec((64, 96), lambda i: (0, 0))],
        out_specs=pl.BlockSpec((BLK, 96), lambda i: (i, 0)),
        out_shape=jax.ShapeDtypeStruct((N_NODES, 96), f32),
    )(acc1, E8, b1.reshape(1, 64), Wall2)
    G2 = P2[:, :80]
    D2 = jnp.pad(P2[:, 80:96], ((0, N_ACC - N_NODES), (0, 0)))

    acc2 = _make_edge_kernel(False)(G2, D2, src, dst)

    out = pl.pallas_call(
        _fin_body,
        grid=(nblk,),
        in_specs=[pl.BlockSpec((NC, BLK, 80), lambda i: (0, i, 0)),
                  pl.BlockSpec((1, 64), lambda i: (0, 0))],
        out_specs=pl.BlockSpec((BLK, 64), lambda i: (i, 0)),
        out_shape=jax.ShapeDtypeStruct((N_NODES, 64), f32),
    )(acc2, b2.reshape(1, 64))
    return out


# speedup vs baseline: 1.8704x; 1.0002x over previous
import functools

import jax
import jax.numpy as jnp
from jax import lax
from jax.experimental import pallas as pl
from jax.experimental.pallas import tpu as pltpu
from jax.experimental.pallas import tpu_sc as plsc

N_NODES = 10000
D_IN = 128
N_EDGES = 320000
E_TOT = N_EDGES + N_NODES
NC, NS, LANES = 2, 16, 16
NW = NC * NS
CHUNK = 128
CHUNKS_PER_TILE = 2 * (-(-E_TOT // (NW * CHUNK * 2)))
E_PAD = NW * CHUNK * CHUNKS_PER_TILE
N_ACC = 10240
ROWS_PER_TILE = N_ACC // NS
BLK = 400


def _bcast_reg(v, k, heads8):
    lane = lax.iota(jnp.int32, 16)
    idx = (lane // 8) + (2 * k) if heads8 else lane * 0
    dn = lax.GatherDimensionNumbers(
        offset_dims=(), collapsed_slice_dims=(0,), start_index_map=(0,))
    return lax.gather(v, idx[:, None], dn, (1,),
                      mode=lax.GatherScatterMode.PROMISE_IN_BOUNDS)


@functools.lru_cache(maxsize=None)
def _make_edge_kernel(heads8):
    mesh = plsc.VectorSubcoreMesh(
        core_axis_name="c", subcore_axis_name="s", num_cores=NC, num_subcores=NS
    )

    @functools.partial(
        pl.kernel,
        out_type=jax.ShapeDtypeStruct((NC, N_ACC, 80), jnp.float32),
        mesh=mesh,
        compiler_params=pltpu.CompilerParams(
            use_tc_tiling_on_sc=False, needs_layout_passes=False),
        scratch_types=[
            pltpu.VMEM((2, CHUNK), jnp.int32),
            pltpu.VMEM((2, CHUNK), jnp.int32),
            pltpu.VMEM((2, CHUNK, 80), jnp.float32),
            pltpu.VMEM((2, CHUNK, 16), jnp.float32),
            pltpu.VMEM((CHUNK, 80), jnp.float32),
            pltpu.VMEM_SHARED((N_ACC, 80), jnp.float32),
            pltpu.SemaphoreType.DMA,
            pltpu.SemaphoreType.DMA,
            pltpu.SemaphoreType.DMA,
            pltpu.SemaphoreType.DMA,
        ],
    )
    def edge_kernel(g_hbm, d_hbm, src_hbm, dst_hbm, acc_hbm,
                    idx_s, idx_d, rows_s, rows_d, msg, acc_sh,
                    sem_s0, sem_s1, sem_d0, sem_d1):
        cid = lax.axis_index("c")
        sid = lax.axis_index("s")
        wid = sid * NC + cid
        sem_s = (sem_s0, sem_s1)
        sem_d = (sem_d0, sem_d1)
        ebase = wid * (CHUNKS_PER_TILE * CHUNK)

        def issue(g, b):
            off = ebase + g * CHUNK
            pltpu.sync_copy(src_hbm.at[pl.ds(off, CHUNK)], idx_s.at[b])
            pltpu.sync_copy(dst_hbm.at[pl.ds(off, CHUNK)], idx_d.at[b])
            pltpu.async_copy(g_hbm.at[idx_s.at[b]], rows_s.at[b], sem_s[b])
            pltpu.async_copy(d_hbm.at[idx_d.at[b]], rows_d.at[b], sem_d[b])

        issue(0, 0)

        def zrow(r, _):
            for k in range(5):
                msg[r, pl.ds(16 * k, 16)] = jnp.zeros((16,), jnp.float32)
            return 0
        lax.fori_loop(0, CHUNK, zrow, 0)
        row0 = sid * ROWS_PER_TILE
        for j in range(ROWS_PER_TILE // CHUNK):
            pltpu.sync_copy(msg, acc_sh.at[pl.ds(row0 + j * CHUNK, CHUNK)])
        plsc.subcore_barrier()

        def pair(i, _):
            for b in range(2):
                g = 2 * i + b
                pltpu.make_async_copy(
                    g_hbm.at[idx_s.at[b]], rows_s.at[b], sem_s[b]).wait()
                pltpu.make_async_copy(
                    d_hbm.at[idx_d.at[b]], rows_d.at[b], sem_d[b]).wait()

                @pl.when(g + 1 < CHUNKS_PER_TILE)
                def _():
                    issue(g + 1, 1 - b)

                rs = rows_s.at[b]

                @plsc.parallel_loop(0, CHUNK, step=1, unroll=4)
                def _edges(e):
                    va = rs[e, pl.ds(64, 16)]
                    vd = rows_d[b, e, pl.ds(0, 16)]
                    z = va + vd
                    z = jnp.maximum(z, z * jnp.float32(0.2))
                    sv = jnp.exp(z)
                    msg[e, pl.ds(64, 16)] = sv
                    for k in range(4):
                        bk = _bcast_reg(sv, k, heads8)
                        msg[e, pl.ds(16 * k, 16)] = rs[e, pl.ds(16 * k, 16)] * bk
                pltpu.sync_copy(msg, acc_sh.at[idx_d.at[b]], add=True)
            return 0
        lax.fori_loop(0, CHUNKS_PER_TILE // 2, pair, 0)
        plsc.subcore_barrier()

        for j in range(ROWS_PER_TILE // CHUNK):
            r = row0 + j * CHUNK
            pltpu.sync_copy(acc_sh.at[pl.ds(r, CHUNK)], msg)
            pltpu.sync_copy(msg, acc_hbm.at[cid, pl.ds(r, CHUNK)])

    return edge_kernel


def _mm_body(x_ref, w_ref, o_ref):
    o_ref[...] = jnp.dot(x_ref[...], w_ref[...],
                         preferred_element_type=jnp.float32)


def _ep1_body(acc_ref, e8_ref, b1_ref, w_ref, o_ref):
    o = acc_ref[0] + acc_ref[1]
    m = o[:, :64]
    den = o[:, 64:72]
    den64 = jnp.dot(den, e8_ref[...], preferred_element_type=jnp.float32)
    hr = jnp.maximum(m / (den64 + jnp.float32(1e-16)) + b1_ref[...], 0.0)
    o_ref[...] = jnp.dot(hr, w_ref[...], preferred_element_type=jnp.float32)


def _fin_body(acc_ref, b2_ref, o_ref):
    o = acc_ref[0] + acc_ref[1]
    logits = o[:, :64] / (o[:, 64:65] + jnp.float32(1e-16)) + b2_ref[...]
    mx = jnp.max(logits, axis=1, keepdims=True)
    z = logits - mx
    o_ref[...] = z - jnp.log(jnp.sum(jnp.exp(z), axis=1, keepdims=True))


def kernel(x, edge_index, W1, a_src1, a_dst1, b1, W2, a_src2, a_dst2, b2):
    f32 = jnp.float32
    As = (jnp.eye(8, dtype=f32)[:, None, :] * a_src1[0][:, :, None]).reshape(64, 8)
    Ad = (jnp.eye(8, dtype=f32)[:, None, :] * a_dst1[0][:, :, None]).reshape(64, 8)
    W1s = W1 @ As
    W1d = W1 @ Ad
    Wall1 = jnp.concatenate(
        [W1, W1s, W1d, W1d, jnp.zeros((D_IN, 8), f32)], axis=1)
    W2s = (W2 @ a_src2[0, 0])[:, None]
    W2d = (W2 @ a_dst2[0, 0])[:, None]
    Wall2 = jnp.concatenate(
        [W2, W2s, jnp.zeros((64, 15), f32), W2d, jnp.zeros((64, 15), f32)],
        axis=1)
    E8 = jnp.repeat(jnp.eye(8, dtype=f32), 8, axis=1)

    loop = jnp.arange(N_NODES, dtype=jnp.int32)
    pad = E_PAD - E_TOT
    pad_dst = N_NODES + (jnp.arange(pad, dtype=jnp.int32) % (N_ACC - N_NODES))
    pad_src = jnp.arange(pad, dtype=jnp.int32) % N_NODES
    src = jnp.concatenate([edge_index[0], loop, pad_src])
    dst = jnp.concatenate([edge_index[1], loop, pad_dst])

    nblk = N_NODES // BLK

    P1 = pl.pallas_call(
        _mm_body,
        grid=(nblk,),
        in_specs=[pl.BlockSpec((BLK, D_IN), lambda i: (i, 0)),
                  pl.BlockSpec((D_IN, 96), lambda i: (0, 0))],
        out_specs=pl.BlockSpec((BLK, 96), lambda i: (i, 0)),
        out_shape=jax.ShapeDtypeStruct((N_NODES, 96), f32),
    )(x, Wall1)
    G1 = P1[:, :80]
    D1 = jnp.pad(P1[:, 80:96], ((0, N_ACC - N_NODES), (0, 0)))

    acc1 = _make_edge_kernel(True)(G1, D1, src, dst)

    P2 = pl.pallas_call(
        _ep1_body,
        grid=(nblk,),
        in_specs=[pl.BlockSpec((NC, BLK, 80), lambda i: (0, i, 0)),
                  pl.BlockSpec((8, 64), lambda i: (0, 0)),
                  pl.BlockSpec((1, 64), lambda i: (0, 0)),
                  pl.BlockSpec((64, 96), lambda i: (0, 0))],
        out_specs=pl.BlockSpec((BLK, 96), lambda i: (i, 0)),
        out_shape=jax.ShapeDtypeStruct((N_NODES, 96), f32),
    )(acc1, E8, b1.reshape(1, 64), Wall2)
    G2 = P2[:, :80]
    D2 = jnp.pad(P2[:, 80:96], ((0, N_ACC - N_NODES), (0, 0)))

    acc2 = _make_edge_kernel(False)(G2, D2, src, dst)

    out = pl.pallas_call(
        _fin_body,
        grid=(nblk,),
        in_specs=[pl.BlockSpec((NC, BLK, 80), lambda i: (0, i, 0)),
                  pl.BlockSpec((1, 64), lambda i: (0, 0))],
        out_specs=pl.BlockSpec((BLK, 64), lambda i: (i, 0)),
        out_shape=jax.ShapeDtypeStruct((N_NODES, 64), f32),
    )(acc2, b2.reshape(1, 64))
    return out


# speedup vs baseline: 2.5744x; 1.3764x over previous
import functools

import jax
import jax.numpy as jnp
from jax import lax
from jax.experimental import pallas as pl
from jax.experimental.pallas import tpu as pltpu
from jax.experimental.pallas import tpu_sc as plsc

N_NODES = 10000
D_IN = 128
N_EDGES = 320000
E_TOT = N_EDGES + N_NODES
NC, NS, LANES = 2, 16, 16
NW = NC * NS
CHUNK = 128
CHUNKS_PER_TILE = 2 * (-(-E_TOT // (NW * CHUNK * 2)))
E_PAD = NW * CHUNK * CHUNKS_PER_TILE
N_ACC = 10240
ROWS_PER_TILE = N_ACC // NS
BLK = 400


def _bcast_reg(v, k, heads8):
    lane = lax.iota(jnp.int32, 16)
    idx = (lane // 8) + (2 * k) if heads8 else lane * 0
    dn = lax.GatherDimensionNumbers(
        offset_dims=(), collapsed_slice_dims=(0,), start_index_map=(0,))
    return lax.gather(v, idx[:, None], dn, (1,),
                      mode=lax.GatherScatterMode.PROMISE_IN_BOUNDS)


@functools.lru_cache(maxsize=None)
def _make_edge_kernel(heads8):
    mesh = plsc.VectorSubcoreMesh(
        core_axis_name="c", subcore_axis_name="s", num_cores=NC, num_subcores=NS
    )

    @functools.partial(
        pl.kernel,
        out_type=jax.ShapeDtypeStruct((NC, N_ACC, 80), jnp.float32),
        mesh=mesh,
        compiler_params=pltpu.CompilerParams(
            use_tc_tiling_on_sc=False, needs_layout_passes=False),
        scratch_types=[
            pltpu.VMEM((CHUNKS_PER_TILE, CHUNK), jnp.int32),
            pltpu.VMEM((CHUNKS_PER_TILE, CHUNK), jnp.int32),
            pltpu.VMEM((2, CHUNK, 80), jnp.float32),
            pltpu.VMEM((2, CHUNK, 16), jnp.float32),
            pltpu.VMEM((2, CHUNK, 80), jnp.float32),
            pltpu.VMEM_SHARED((N_ACC, 80), jnp.float32),
            pltpu.SemaphoreType.DMA,
            pltpu.SemaphoreType.DMA,
            pltpu.SemaphoreType.DMA,
            pltpu.SemaphoreType.DMA,
            pltpu.SemaphoreType.DMA,
            pltpu.SemaphoreType.DMA,
        ],
    )
    def edge_kernel(g_hbm, d_hbm, src2_hbm, dst2_hbm, acc_hbm,
                    idx_sa, idx_da, rows_s, rows_d, msg, acc_sh,
                    sem_s0, sem_s1, sem_d0, sem_d1, sem_c0, sem_c1):
        cid = lax.axis_index("c")
        sid = lax.axis_index("s")
        wid = sid * NC + cid
        sem_s = (sem_s0, sem_s1)
        sem_d = (sem_d0, sem_d1)
        sem_c = (sem_c0, sem_c1)

        cbase = wid * CHUNKS_PER_TILE
        pltpu.sync_copy(src2_hbm.at[pl.ds(cbase, CHUNKS_PER_TILE)], idx_sa)
        pltpu.sync_copy(dst2_hbm.at[pl.ds(cbase, CHUNKS_PER_TILE)], idx_da)

        def issue(g, b):
            pltpu.async_copy(g_hbm.at[idx_sa.at[g]], rows_s.at[b], sem_s[b])
            pltpu.async_copy(d_hbm.at[idx_da.at[g]], rows_d.at[b], sem_d[b])

        issue(0, 0)

        z0 = msg.at[0]

        def zrow(r, _):
            for k in range(5):
                z0[r, pl.ds(16 * k, 16)] = jnp.zeros((16,), jnp.float32)
            return 0
        lax.fori_loop(0, CHUNK, zrow, 0)
        row0 = sid * ROWS_PER_TILE
        for j in range(ROWS_PER_TILE // CHUNK):
            pltpu.sync_copy(z0, acc_sh.at[pl.ds(row0 + j * CHUNK, CHUNK)])
        plsc.subcore_barrier()

        def pair(i, _):
            for b in range(2):
                g = 2 * i + b
                pltpu.make_async_copy(
                    g_hbm.at[idx_sa.at[g]], rows_s.at[b], sem_s[b]).wait()
                pltpu.make_async_copy(
                    d_hbm.at[idx_da.at[g]], rows_d.at[b], sem_d[b]).wait()

                @pl.when(g + 1 < CHUNKS_PER_TILE)
                def _():
                    issue(g + 1, 1 - b)

                @pl.when(g >= 2)
                def _():
                    pltpu.make_async_copy(
                        msg.at[b], acc_sh.at[idx_da.at[g]], sem_c[b]).wait()

                rs = rows_s.at[b]
                mb = msg.at[b]

                @plsc.parallel_loop(0, CHUNK, step=1, unroll=4)
                def _edges(e):
                    va = rs[e, pl.ds(64, 16)]
                    vd = rows_d[b, e, pl.ds(0, 16)]
                    z = va + vd
                    z = jnp.maximum(z, z * jnp.float32(0.2))
                    sv = jnp.exp(z)
                    mb[e, pl.ds(64, 16)] = sv
                    for k in range(4):
                        bk = _bcast_reg(sv, k, heads8)
                        mb[e, pl.ds(16 * k, 16)] = rs[e, pl.ds(16 * k, 16)] * bk
                pltpu.async_copy(
                    mb, acc_sh.at[idx_da.at[g]], sem_c[b], add=True)
            return 0
        lax.fori_loop(0, CHUNKS_PER_TILE // 2, pair, 0)
        for b in range(2):
            pltpu.make_async_copy(
                msg.at[b], acc_sh.at[idx_da.at[b]], sem_c[b]).wait()
        plsc.subcore_barrier()

        for j in range(ROWS_PER_TILE // CHUNK):
            r = row0 + j * CHUNK
            pltpu.sync_copy(acc_sh.at[pl.ds(r, CHUNK)], z0)
            pltpu.sync_copy(z0, acc_hbm.at[cid, pl.ds(r, CHUNK)])

    return edge_kernel


def _mm_body(x_ref, w_ref, o_ref):
    o_ref[...] = jnp.dot(x_ref[...], w_ref[...],
                         preferred_element_type=jnp.float32)


def _ep1_body(acc_ref, e8_ref, b1_ref, w_ref, o_ref):
    o = acc_ref[0] + acc_ref[1]
    m = o[:, :64]
    den = o[:, 64:72]
    den64 = jnp.dot(den, e8_ref[...], preferred_element_type=jnp.float32)
    hr = jnp.maximum(m / (den64 + jnp.float32(1e-16)) + b1_ref[...], 0.0)
    o_ref[...] = jnp.dot(hr, w_ref[...], preferred_element_type=jnp.float32)


def _fin_body(acc_ref, b2_ref, o_ref):
    o = acc_ref[0] + acc_ref[1]
    logits = o[:, :64] / (o[:, 64:65] + jnp.float32(1e-16)) + b2_ref[...]
    mx = jnp.max(logits, axis=1, keepdims=True)
    z = logits - mx
    o_ref[...] = z - jnp.log(jnp.sum(jnp.exp(z), axis=1, keepdims=True))


def kernel(x, edge_index, W1, a_src1, a_dst1, b1, W2, a_src2, a_dst2, b2):
    f32 = jnp.float32
    As = (jnp.eye(8, dtype=f32)[:, None, :] * a_src1[0][:, :, None]).reshape(64, 8)
    Ad = (jnp.eye(8, dtype=f32)[:, None, :] * a_dst1[0][:, :, None]).reshape(64, 8)
    W1s = W1 @ As
    W1d = W1 @ Ad
    Wall1 = jnp.concatenate(
        [W1, W1s, W1d, W1d, jnp.zeros((D_IN, 8), f32)], axis=1)
    W2s = (W2 @ a_src2[0, 0])[:, None]
    W2d = (W2 @ a_dst2[0, 0])[:, None]
    Wall2 = jnp.concatenate(
        [W2, W2s, jnp.zeros((64, 15), f32), W2d, jnp.zeros((64, 15), f32)],
        axis=1)
    E8 = jnp.repeat(jnp.eye(8, dtype=f32), 8, axis=1)

    loop = jnp.arange(N_NODES, dtype=jnp.int32)
    pad = E_PAD - E_TOT
    pad_dst = N_NODES + (jnp.arange(pad, dtype=jnp.int32) % (N_ACC - N_NODES))
    pad_src = jnp.arange(pad, dtype=jnp.int32) % N_NODES
    src = jnp.concatenate([edge_index[0], loop, pad_src]).reshape(-1, CHUNK)
    dst = jnp.concatenate([edge_index[1], loop, pad_dst]).reshape(-1, CHUNK)

    nblk = N_NODES // BLK

    P1 = pl.pallas_call(
        _mm_body,
        grid=(nblk,),
        in_specs=[pl.BlockSpec((BLK, D_IN), lambda i: (i, 0)),
                  pl.BlockSpec((D_IN, 96), lambda i: (0, 0))],
        out_specs=pl.BlockSpec((BLK, 96), lambda i: (i, 0)),
        out_shape=jax.ShapeDtypeStruct((N_NODES, 96), f32),
    )(x, Wall1)
    G1 = P1[:, :80]
    D1 = jnp.pad(P1[:, 80:96], ((0, N_ACC - N_NODES), (0, 0)))

    acc1 = _make_edge_kernel(True)(G1, D1, src, dst)

    P2 = pl.pallas_call(
        _ep1_body,
        grid=(nblk,),
        in_specs=[pl.BlockSpec((NC, BLK, 80), lambda i: (0, i, 0)),
                  pl.BlockSpec((8, 64), lambda i: (0, 0)),
                  pl.BlockSpec((1, 64), lambda i: (0, 0)),
                  pl.BlockSpec((64, 96), lambda i: (0, 0))],
        out_specs=pl.BlockSpec((BLK, 96), lambda i: (i, 0)),
        out_shape=jax.ShapeDtypeStruct((N_NODES, 96), f32),
    )(acc1, E8, b1.reshape(1, 64), Wall2)
    G2 = P2[:, :80]
    D2 = jnp.pad(P2[:, 80:96], ((0, N_ACC - N_NODES), (0, 0)))

    acc2 = _make_edge_kernel(False)(G2, D2, src, dst)

    out = pl.pallas_call(
        _fin_body,
        grid=(nblk,),
        in_specs=[pl.BlockSpec((NC, BLK, 80), lambda i: (0, i, 0)),
                  pl.BlockSpec((1, 64), lambda i: (0, 0))],
        out_specs=pl.BlockSpec((BLK, 64), lambda i: (i, 0)),
        out_shape=jax.ShapeDtypeStruct((N_NODES, 64), f32),
    )(acc2, b2.reshape(1, 64))
    return out
